# baseline (speedup 1.0000x reference)
import functools

import jax
import jax.numpy as jnp
from jax import lax
from jax.experimental import pallas as pl
from jax.experimental.pallas import tpu as pltpu
from jax.experimental.pallas import tpu_sc as plsc

_NC = 2
_NS = 16
_LANES = 16


def _build(B, L, V, D):
    DPC = D // _NC
    BPT = B // _NS
    SEG = 65536
    assert _NS * SEG >= V
    NRUN = SEG // 128
    CL = 8
    NCHK = L // CL
    NBLK = (NCHK - 1) // 2

    mesh = plsc.VectorSubcoreMesh(core_axis_name="c", subcore_axis_name="s")

    scratch = [
        pltpu.VMEM_SHARED((_NS * SEG,), jnp.float32),
        pltpu.VMEM((L * BPT,), jnp.int32),
        pltpu.VMEM((CL * BPT,), jnp.float32),
        pltpu.VMEM((CL * BPT,), jnp.float32),
        pltpu.VMEM((CL, BPT), jnp.float32),
        pltpu.VMEM((256,), jnp.float32),
        pltpu.SemaphoreType.DMA,
        pltpu.SemaphoreType.DMA,
        pltpu.SemaphoreType.DMA,
        pltpu.SemaphoreType.DMA,
        pltpu.SemaphoreType.DMA,
    ]

    @functools.partial(
        pl.kernel,
        out_type=jax.ShapeDtypeStruct((L, D, B), jnp.float32),
        mesh=mesh,
        scratch_types=scratch,
    )
    def run(tok_hbm, tab_hbm, pos_hbm, out_hbm,
            row_sp, tok1d, gat0, gat1, wb, pos_v,
            ssem, tsem, gsem0, gsem1, osem):
        c = lax.axis_index("c")
        s = lax.axis_index("s")
        b0 = s * BPT
        d_base = c * DPC
        v0 = s * SEG
        gats = (gat0, gat1)
        gsems = (gsem0, gsem1)

        def fire_stage(dd):
            def one(j, c2):
                pltpu.async_copy(tab_hbm.at[dd, pl.ds(v0 + j * 128, 128)],
                                 row_sp.at[pl.ds(v0 + j * 128, 128)], ssem)
                return c2
            lax.fori_loop(0, NRUN, one, 0)

        def drain_stage(dd):
            def one(j, c2):
                pltpu.make_async_copy(
                    tab_hbm.at[dd, pl.ds(v0 + j * 128, 128)],
                    row_sp.at[pl.ds(v0 + j * 128, 128)], ssem).wait()
                return c2
            lax.fori_loop(0, NRUN, one, 0)

        fire_stage(d_base)

        def stage_tok(l, c2):
            pltpu.async_copy(tok_hbm.at[pl.ds(l * B + b0, BPT)],
                             tok1d.at[pl.ds(l * BPT, BPT)], tsem)
            return c2

        lax.fori_loop(0, L, stage_tok, 0)

        def drain_tok(l, c2):
            pltpu.make_async_copy(tok_hbm.at[pl.ds(l * B + b0, BPT)],
                                  tok1d.at[pl.ds(l * BPT, BPT)], tsem).wait()
            return c2

        lax.fori_loop(0, L, drain_tok, 0)

        def fire_gather(n, g):
            pltpu.async_copy(
                row_sp.at[tok1d.at[pl.ds(n * CL * BPT, CL * BPT)]],
                gats[g], gsems[g])

        def wait_gather(n, g):
            pltpu.make_async_copy(
                row_sp.at[tok1d.at[pl.ds(n * CL * BPT, CL * BPT)]],
                gats[g], gsems[g]).wait()

        def wb_push(n, dd):
            for li in range(CL):
                pltpu.async_copy(
                    wb.at[li], out_hbm.at[n * CL + li, dd, pl.ds(b0, BPT)],
                    osem)

        def wb_drain(n, dd):
            for li in range(CL):
                pltpu.make_async_copy(
                    wb.at[li], out_hbm.at[n * CL + li, dd, pl.ds(b0, BPT)],
                    osem).wait()

        def add_chunk(g, pv16, li0, first):
            @pl.when(jnp.logical_not(first))
            def _drain():
                wb_drain(0, 0)
            for li in range(CL):
                pv = jnp.full((_LANES,), pv16[li0 + li], dtype=jnp.float32)
                for j in range(BPT // _LANES):
                    sl = pl.ds((li * BPT) + j * _LANES, _LANES)
                    wb[li, pl.ds(j * _LANES, _LANES)] = gats[g][sl] + pv

        def phase(p, carry):
            dd = d_base + p
            drain_stage(dd)
            plsc.subcore_barrier()
            pltpu.sync_copy(pos_hbm.at[pl.ds(dd * L, L)],
                            pos_v.at[pl.ds(0, L)])
            fire_gather(0, 0)
            fire_gather(1, 1)

            def block(k, c2):
                pv16 = pos_v[pl.ds(k * _LANES, _LANES)]
                for i in range(2):
                    n = 2 * k + i
                    wait_gather(n, i)
                    add_chunk(i, pv16, i * CL, (p == 0) & (n == 0))
                    wb_push(n, dd)

                    @pl.when(n + 2 < NCHK)
                    def _ahead():
                        fire_gather(n + 2, i)
                return c2

            lax.fori_loop(0, NBLK, block, 0)
            wait_gather(NCHK - 1, 0)
            plsc.subcore_barrier()

            @pl.when(p + 1 < DPC)
            def _stage():
                fire_stage(dd + 1)

            pv16 = pos_v[pl.ds(L - _LANES, _LANES)]
            add_chunk(0, pv16, _LANES - CL, False)
            wb_push(NCHK - 1, dd)
            return carry

        lax.fori_loop(0, DPC, phase, 0)
        wb_drain(0, 0)

    return run


def kernel(tokens, token_table, pos_emb):
    B, L = tokens.shape
    V, D = token_table.shape
    run = _build(B, L, V, D)
    out_t = run(tokens.T.reshape(L * B), token_table.T,
                pos_emb.T.reshape(D * L))
    return jnp.transpose(out_t, (2, 0, 1))


# speedup vs baseline: 1.2835x; 1.2835x over previous
import functools

import jax
import jax.numpy as jnp
from jax import lax
from jax.experimental import pallas as pl
from jax.experimental.pallas import tpu as pltpu
from jax.experimental.pallas import tpu_sc as plsc

_NC = 2
_NS = 16
_LANES = 16


def _build(B, L, V, D):
    T = B * L
    NW = _NC * _NS
    per_w = T // NW
    C = L
    n_chunks = per_w // C
    PW = 128

    mesh = plsc.VectorSubcoreMesh(core_axis_name="c", subcore_axis_name="s")

    scratch = (
        [pltpu.VMEM((C,), jnp.int32) for _ in range(2)]
        + [pltpu.VMEM((C,), jnp.int32) for _ in range(2)]
        + [pltpu.VMEM((C, PW), jnp.float32) for _ in range(2)]
        + [pltpu.VMEM((C, PW), jnp.float32) for _ in range(2)]
        + [pltpu.VMEM((C, D), jnp.float32)]
        + [pltpu.SemaphoreType.DMA for _ in range(8)]
    )

    @functools.partial(
        pl.kernel,
        out_type=jax.ShapeDtypeStruct((T, PW), jnp.float32),
        mesh=mesh,
        scratch_types=scratch,
    )
    def run(tok_hbm, tk2_hbm, tab_hbm, pos_hbm, out_hbm, *scr):
        tok = scr[0:2]
        tk2 = scr[2:4]
        gat = scr[4:6]
        wb = scr[6:8]
        pos_v = scr[8]
        tsem = (scr[9], scr[10])
        isem = (scr[11], scr[12])
        gsem = (scr[13], scr[14])
        osem = (scr[15], scr[16])

        wid = lax.axis_index("s") * _NC + lax.axis_index("c")
        base = wid * per_w
        pltpu.sync_copy(pos_hbm, pos_v)

        def fire_idx(g, b):
            sl = pl.ds(base + g * C, C)
            pltpu.async_copy(tok_hbm.at[sl], tok[b], tsem[b])
            pltpu.async_copy(tk2_hbm.at[sl], tk2[b], isem[b])

        def drain_idx(g, b):
            sl = pl.ds(base + g * C, C)
            pltpu.make_async_copy(tok_hbm.at[sl], tok[b], tsem[b]).wait()
            pltpu.make_async_copy(tk2_hbm.at[sl], tk2[b], isem[b]).wait()

        def fire_gather(b):
            pltpu.async_copy(tab_hbm.at[tk2[b]], gat[b], gsem[b])

        def drain_gather(b):
            pltpu.make_async_copy(tab_hbm.at[tk2[b]], gat[b], gsem[b]).wait()

        def fire_wb(g, b):
            pltpu.async_copy(wb[b], out_hbm.at[pl.ds(base + g * C, C)],
                             osem[b])

        def drain_wb(g, b):
            pltpu.make_async_copy(wb[b], out_hbm.at[pl.ds(base + g * C, C)],
                                  osem[b]).wait()

        fire_idx(0, 0)
        fire_idx(1, 1)
        drain_idx(0, 0)
        fire_gather(0)

        def visit(g, b):
            @pl.when(g + 1 < n_chunks)
            def _next():
                drain_idx(g + 1, b ^ 1)
                fire_gather(b ^ 1)
            drain_gather(b)

            @pl.when(g + 2 < n_chunks)
            def _i():
                fire_idx(g + 2, b)

            @pl.when(g >= 2)
            def _w():
                drain_wb(g - 2, b)

            def add_block(k, c2):
                tv = tok[b][pl.ds(k * _LANES, _LANES)]
                for li in range(_LANES):
                    r = k * _LANES + li
                    off = (tv[li] & 1) * D
                    for j in range(D // _LANES):
                        wb[b][r, pl.ds(j * _LANES, _LANES)] = (
                            gat[b][r, pl.ds(off + j * _LANES, _LANES)]
                            + pos_v[r, pl.ds(j * _LANES, _LANES)])
                return c2

            lax.fori_loop(0, C // _LANES, add_block, 0)
            fire_wb(g, b)

        def outer(m, carry):
            for bo in range(2):
                visit(m * 2 + bo, bo)
            return carry

        lax.fori_loop(0, n_chunks // 2, outer, 0)
        drain_wb(n_chunks - 2, 0)
        drain_wb(n_chunks - 1, 1)

    return run


def kernel(tokens, token_table, pos_emb):
    B, L = tokens.shape
    V, D = token_table.shape
    run = _build(B, L, V, D)
    flat = tokens.reshape(B * L)
    out = run(flat, flat >> 1, token_table.reshape(V // 2, 2 * D), pos_emb)
    return out[:, :D].reshape(B, L, D)
